# baseline (speedup 1.0000x reference)
import functools

import jax
import jax.numpy as jnp
from jax import lax
from jax.experimental import pallas as pl
from jax.experimental.pallas import tpu as pltpu
from jax.experimental.pallas import tpu_sc as plsc

_N = 10000
_H = 128
_HH = 64
_E = 320000

_NC = 2
_NS = 16
_NW = _NC * _NS
_CHUNK = 128
_CH = 80
_EPW = _CHUNK * _CH
_EPAD = _NW * _EPW
_NPAD = 10240
_RPT = _NPAD // _NS

_mesh = plsc.VectorSubcoreMesh(core_axis_name="c", subcore_axis_name="s")

_sc_params = pltpu.CompilerParams(use_tc_tiling_on_sc=False)


@functools.partial(
    pl.kernel,
    out_type=jax.ShapeDtypeStruct((_NC, _NPAD, 16), jnp.float32),
    mesh=_mesh,
    scratch_types=[
        pltpu.VMEM((_CH, _CHUNK), jnp.int32),
        pltpu.VMEM((_CHUNK, 16), jnp.float32),
        pltpu.VMEM_SHARED((_NPAD, 16), jnp.float32),
    ],
    compiler_params=_sc_params,
)
def _sc_deg(dst_hbm, ones_hbm, zeros_hbm, out_hbm, dst_v, ones_v, acc):
    c = lax.axis_index("c")
    s = lax.axis_index("s")
    w = c * _NS + s
    pltpu.sync_copy(dst_hbm.at[w], dst_v)
    pltpu.sync_copy(ones_hbm, ones_v)
    pltpu.sync_copy(zeros_hbm, acc.at[pl.ds(s * _RPT, _RPT)])
    plsc.subcore_barrier()

    @pl.loop(0, _CH)
    def _(j):
        pltpu.sync_copy(ones_v, acc.at[dst_v.at[j]], add=True)

    plsc.subcore_barrier()
    pltpu.sync_copy(acc.at[pl.ds(s * _RPT, _RPT)],
                    out_hbm.at[c, pl.ds(s * _RPT, _RPT)])


@functools.partial(
    pl.kernel,
    out_type=jax.ShapeDtypeStruct((_NC, 2, _NPAD, _HH), jnp.float32),
    mesh=_mesh,
    scratch_types=[
        pltpu.VMEM((_CH, _CHUNK), jnp.int32),
        pltpu.VMEM((_CH, _CHUNK), jnp.int32),
        pltpu.VMEM((4, _CHUNK, _HH), jnp.float32),
        pltpu.VMEM_SHARED((_NPAD, _HH), jnp.float32),
        [pltpu.SemaphoreType.DMA] * 4,
        [pltpu.SemaphoreType.DMA] * 4,
    ],
    compiler_params=_sc_params,
)
def _sc_agg(y_hbm, src_hbm, dst_hbm, zeros_hbm, out_hbm,
            src_v, dst_v, bufs, acc, gsems, ssems):
    c = lax.axis_index("c")
    s = lax.axis_index("s")
    w = c * _NS + s
    pltpu.sync_copy(src_hbm.at[w], src_v)
    pltpu.sync_copy(dst_hbm.at[w], dst_v)

    for h in range(2):
        yh = y_hbm.at[h]
        pltpu.sync_copy(zeros_hbm, acc.at[pl.ds(s * _RPT, _RPT)])
        plsc.subcore_barrier()

        pltpu.async_copy(yh.at[src_v.at[0]], bufs.at[0], gsems[0])
        pltpu.async_copy(yh.at[src_v.at[1]], bufs.at[1], gsems[1])

        @pl.loop(0, _CH, step=4)
        def _(j):
            for b in range(4):
                m = j + b
                b2 = (b + 2) % 4
                pltpu.make_async_copy(yh.at[src_v.at[0]], bufs.at[b],
                                      gsems[b]).wait()
                pltpu.async_copy(bufs.at[b], acc.at[dst_v.at[m]], ssems[b],
                                 add=True)

                @pl.when(m + 2 < _CH)
                def _():
                    @pl.when(m - 2 >= 0)
                    def _():
                        pltpu.make_async_copy(
                            bufs.at[b2], acc.at[dst_v.at[0]], ssems[b2]).wait()

                    pltpu.async_copy(yh.at[src_v.at[m + 2]], bufs.at[b2],
                                     gsems[b2])

        for b in range(4):
            pltpu.make_async_copy(bufs.at[b], acc.at[dst_v.at[0]],
                                  ssems[b]).wait()

        plsc.subcore_barrier()
        pltpu.sync_copy(acc.at[pl.ds(s * _RPT, _RPT)],
                        out_hbm.at[c, h, pl.ds(s * _RPT, _RPT)])


_BLK = 2000
_NG = _N // _BLK


def _tc_first_body(degp_ref, x_ref, w_ref, y_ref):
    deg = degp_ref[0, : _N, 0:1] + degp_ref[1, : _N, 0:1] + 1.0
    dinv = lax.rsqrt(deg)
    xw = jnp.dot(x_ref[...], w_ref[...])
    y = xw * dinv
    y_ref[0] = y[:, :_HH]
    y_ref[1] = y[:, _HH:]


def _tc_h_body(degp_ref, p_ref, y_ref, b_ref, wn_ref, h_ref, yn_ref):
    deg = degp_ref[0, :, 0:1] + degp_ref[1, :, 0:1] + 1.0
    dinv = lax.rsqrt(deg)
    h0 = jnp.maximum(
        (p_ref[0, 0] + p_ref[1, 0] + y_ref[0]) * dinv + b_ref[:, :_HH], 0.0)
    h1 = jnp.maximum(
        (p_ref[0, 1] + p_ref[1, 1] + y_ref[1]) * dinv + b_ref[:, _HH:], 0.0)
    h_ref[...] = jnp.concatenate([h0, h1], axis=1)
    yn = (jnp.dot(h0, wn_ref[:_HH, :]) + jnp.dot(h1, wn_ref[_HH:, :])) * dinv
    yn_ref[0] = yn[:, :_HH]
    yn_ref[1] = yn[:, _HH:]


def _tc_pool_body(h_ref, gwr_ref, gb_ref, r_ref, s_ref):
    i = pl.program_id(0)
    h = h_ref[...]
    g = jnp.sum(h * gwr_ref[...], axis=1, keepdims=True) + gb_ref[0, 0]
    e = jnp.exp(1.0 / (1.0 + jnp.exp(-g)))
    pv = jnp.sum(e * h, axis=0, keepdims=True)
    ps = jnp.sum(e)

    @pl.when(i == 0)
    def _():
        r_ref[...] = pv
        s_ref[0] = ps

    @pl.when(i > 0)
    def _():
        r_ref[...] += pv
        s_ref[0] += ps

    @pl.when(i == _NG - 1)
    def _():
        r_ref[...] = r_ref[...] / s_ref[0]


_tc_first = pl.pallas_call(
    _tc_first_body,
    out_shape=jax.ShapeDtypeStruct((2, _N, _HH), jnp.float32),
)

_tc_h = pl.pallas_call(
    _tc_h_body,
    grid=(_NG,),
    in_specs=[
        pl.BlockSpec((2, _BLK, 16), lambda i: (0, i, 0)),
        pl.BlockSpec((2, 2, _BLK, _HH), lambda i: (0, 0, i, 0)),
        pl.BlockSpec((2, _BLK, _HH), lambda i: (0, i, 0)),
        pl.BlockSpec((1, _H), lambda i: (0, 0)),
        pl.BlockSpec((_H, _H), lambda i: (0, 0)),
    ],
    out_specs=[
        pl.BlockSpec((_BLK, _H), lambda i: (i, 0)),
        pl.BlockSpec((2, _BLK, _HH), lambda i: (0, i, 0)),
    ],
    out_shape=[
        jax.ShapeDtypeStruct((_N, _H), jnp.float32),
        jax.ShapeDtypeStruct((2, _N, _HH), jnp.float32),
    ],
)

_tc_pool = pl.pallas_call(
    _tc_pool_body,
    grid=(_NG,),
    in_specs=[
        pl.BlockSpec((_BLK, _H), lambda i: (i, 0)),
        pl.BlockSpec((1, _H), lambda i: (0, 0)),
        pl.BlockSpec((1, 1), lambda i: (0, 0)),
    ],
    out_specs=pl.BlockSpec((1, _H), lambda i: (0, 0)),
    out_shape=jax.ShapeDtypeStruct((1, _H), jnp.float32),
    scratch_shapes=[pltpu.SMEM((1,), jnp.float32)],
)


@jax.jit
def kernel(x, edge_index, W1, b1, W2, b2, W3, b3, gW1, gb1, gW2, gb2, gW3, gb3):
    src = edge_index[0]
    dst = edge_index[1]
    npad = _EPAD - _E
    pad_dst = _N + (jnp.arange(npad, dtype=jnp.int32) % (_NPAD - _N))
    pad_src = jnp.arange(npad, dtype=jnp.int32) % _N
    src3 = jnp.concatenate([src, pad_src])
    dst3 = jnp.concatenate([dst, pad_dst])
    src3 = src3.reshape(_NW, _CH, _CHUNK)
    dst3 = dst3.reshape(_NW, _CH, _CHUNK)

    ones16 = jnp.ones((_CHUNK, 16), jnp.float32)
    zeros16 = jnp.zeros((_RPT, 16), jnp.float32)
    zerosH = jnp.zeros((_RPT, _HH), jnp.float32)

    degp = _sc_deg(dst3, ones16, zeros16)

    y1 = _tc_first(degp, x, W1)

    bs = jnp.stack([b1.reshape(1, _H), b2.reshape(1, _H), b3.reshape(1, _H)])
    gws = jnp.stack([gW1.reshape(1, _H), gW2.reshape(1, _H), gW3.reshape(1, _H)])
    gbs = jnp.stack([gb1.reshape(1, 1), gb2.reshape(1, 1), gb3.reshape(1, 1)])
    wns = jnp.stack([W2, W3, W3])

    def step(y, params):
        b, gwr, gb, wn = params
        p = _sc_agg(y, src3, dst3, zerosH)
        h, y_next = _tc_h(degp, p, y, b, wn)
        r = _tc_pool(h, gwr, gb)
        return y_next, r

    _, rs = lax.scan(step, y1, (bs, gws, gbs, wns))
    return rs.reshape(1, 3 * _H)


# speedup vs baseline: 1.1001x; 1.1001x over previous
import functools

import jax
import jax.numpy as jnp
from jax import lax
from jax.experimental import pallas as pl
from jax.experimental.pallas import tpu as pltpu
from jax.experimental.pallas import tpu_sc as plsc

_N = 10000
_H = 128
_HH = 64
_E = 320000

_NC = 2
_NS = 16
_NW = _NC * _NS
_CHUNK = 128
_CH = 80
_EPW = _CHUNK * _CH
_EPAD = _NW * _EPW
_NPAD = 10240
_RPT = _NPAD // _NS

_mesh = plsc.VectorSubcoreMesh(core_axis_name="c", subcore_axis_name="s")

_sc_params = pltpu.CompilerParams(use_tc_tiling_on_sc=False)


@functools.partial(
    pl.kernel,
    out_type=jax.ShapeDtypeStruct((_NC, _NPAD, 16), jnp.float32),
    mesh=_mesh,
    scratch_types=[
        pltpu.VMEM((_CH, _CHUNK), jnp.int32),
        pltpu.VMEM((_CHUNK, 16), jnp.float32),
        pltpu.VMEM_SHARED((_NPAD, 16), jnp.float32),
    ],
    compiler_params=_sc_params,
)
def _sc_deg(dst_hbm, ones_hbm, zeros_hbm, out_hbm, dst_v, ones_v, acc):
    c = lax.axis_index("c")
    s = lax.axis_index("s")
    w = c * _NS + s
    pltpu.sync_copy(dst_hbm.at[w], dst_v)
    pltpu.sync_copy(ones_hbm, ones_v)
    pltpu.sync_copy(zeros_hbm, acc.at[pl.ds(s * _RPT, _RPT)])
    plsc.subcore_barrier()

    @pl.loop(0, _CH)
    def _(j):
        pltpu.sync_copy(ones_v, acc.at[dst_v.at[j]], add=True)

    plsc.subcore_barrier()
    pltpu.sync_copy(acc.at[pl.ds(s * _RPT, _RPT)],
                    out_hbm.at[c, pl.ds(s * _RPT, _RPT)])


@functools.partial(
    pl.kernel,
    out_type=jax.ShapeDtypeStruct((_NC, 2, _NPAD, _HH), jnp.float32),
    mesh=_mesh,
    scratch_types=[
        pltpu.VMEM((_CH, _CHUNK), jnp.int32),
        pltpu.VMEM((_CH, _CHUNK), jnp.int32),
        pltpu.VMEM((4, _CHUNK, _HH), jnp.float32),
        pltpu.VMEM_SHARED((_NPAD, _HH), jnp.float32),
        [pltpu.SemaphoreType.DMA] * 4,
        [pltpu.SemaphoreType.DMA] * 4,
    ],
    compiler_params=_sc_params,
)
def _sc_agg(y_hbm, src_hbm, dst_hbm, zeros_hbm, out_hbm,
            src_v, dst_v, bufs, acc, gsems, ssems):
    c = lax.axis_index("c")
    s = lax.axis_index("s")
    w = c * _NS + s
    pltpu.sync_copy(src_hbm.at[w], src_v)
    pltpu.sync_copy(dst_hbm.at[w], dst_v)

    for h in range(2):
        yh = y_hbm.at[h]
        pltpu.sync_copy(zeros_hbm, acc.at[pl.ds(s * _RPT, _RPT)])
        plsc.subcore_barrier()

        pltpu.async_copy(yh.at[src_v.at[0]], bufs.at[0], gsems[0])
        pltpu.async_copy(yh.at[src_v.at[1]], bufs.at[1], gsems[1])

        @pl.loop(0, _CH, step=4)
        def _(j):
            for b in range(4):
                m = j + b
                b2 = (b + 2) % 4
                pltpu.make_async_copy(yh.at[src_v.at[0]], bufs.at[b],
                                      gsems[b]).wait()
                pltpu.async_copy(bufs.at[b], acc.at[dst_v.at[m]], ssems[b],
                                 add=True)

                @pl.when(m + 2 < _CH)
                def _():
                    @pl.when(m - 2 >= 0)
                    def _():
                        pltpu.make_async_copy(
                            bufs.at[b2], acc.at[dst_v.at[0]], ssems[b2]).wait()

                    pltpu.async_copy(yh.at[src_v.at[m + 2]], bufs.at[b2],
                                     gsems[b2])

        for b in range(4):
            pltpu.make_async_copy(bufs.at[b], acc.at[dst_v.at[0]],
                                  ssems[b]).wait()

        plsc.subcore_barrier()
        pltpu.sync_copy(acc.at[pl.ds(s * _RPT, _RPT)],
                        out_hbm.at[c, h, pl.ds(s * _RPT, _RPT)])


_NP = _N // 2
_NPP = _NPAD // 2
_BLKP = 1000
_NG = _NP // _BLKP


def _tc_dinv_body(degp_ref, dn_ref, db_ref):
    deg = degp_ref[0, : _N, 0:1] + degp_ref[1, : _N, 0:1] + 1.0
    dn = lax.rsqrt(deg)
    dn_ref[...] = dn
    db_ref[...] = jnp.broadcast_to(dn, (_N, _HH))


def _tc_first_body(dn_ref, x_ref, w_ref, y_ref):
    y = jnp.dot(x_ref[...], w_ref[...]) * dn_ref[...]
    y_ref[0] = y[:, :_HH]
    y_ref[1] = y[:, _HH:]


def _tc_h_body(dpk_ref, p_ref, y_ref, bpk_ref, d_ref, h_ref, yn_ref):
    dpk = dpk_ref[...]
    h0 = jnp.maximum(
        (p_ref[0, 0] + p_ref[1, 0] + y_ref[0]) * dpk + bpk_ref[0:1, :], 0.0)
    h1 = jnp.maximum(
        (p_ref[0, 1] + p_ref[1, 1] + y_ref[1]) * dpk + bpk_ref[1:2, :], 0.0)
    h_ref[0] = h0
    h_ref[1] = h1
    yn_ref[0] = (jnp.dot(h0, d_ref[0, 0]) + jnp.dot(h1, d_ref[1, 0])) * dpk
    yn_ref[1] = (jnp.dot(h0, d_ref[0, 1]) + jnp.dot(h1, d_ref[1, 1])) * dpk


def _tc_pool_body(h_ref, gwpk_ref, gb_ref, r_ref, v_ref, s_ref):
    i = pl.program_id(0)
    h0 = h_ref[0]
    h1 = h_ref[1]
    t = h0 * gwpk_ref[0:1, :] + h1 * gwpk_ref[1:2, :]
    g_lo = jnp.sum(t[:, :_HH], axis=1, keepdims=True) + gb_ref[0, 0]
    g_hi = jnp.sum(t[:, _HH:], axis=1, keepdims=True) + gb_ref[0, 0]
    e_lo = jnp.exp(1.0 / (1.0 + jnp.exp(-g_lo)))
    e_hi = jnp.exp(1.0 / (1.0 + jnp.exp(-g_hi)))
    epk = jnp.concatenate([jnp.broadcast_to(e_lo, (_BLKP, _HH)),
                           jnp.broadcast_to(e_hi, (_BLKP, _HH))], axis=1)
    pv = jnp.concatenate([jnp.sum(epk * h0, axis=0, keepdims=True),
                          jnp.sum(epk * h1, axis=0, keepdims=True)], axis=0)
    ps = jnp.sum(e_lo) + jnp.sum(e_hi)

    @pl.when(i == 0)
    def _():
        v_ref[...] = pv
        s_ref[0] = ps

    @pl.when(i > 0)
    def _():
        v_ref[...] += pv
        s_ref[0] += ps

    @pl.when(i == _NG - 1)
    def _():
        v = v_ref[...] / s_ref[0]
        r_ref[...] = jnp.concatenate(
            [v[0:1, :_HH] + v[0:1, _HH:], v[1:2, :_HH] + v[1:2, _HH:]],
            axis=1)


_tc_dinv = pl.pallas_call(
    _tc_dinv_body,
    out_shape=[
        jax.ShapeDtypeStruct((_N, 1), jnp.float32),
        jax.ShapeDtypeStruct((_N, _HH), jnp.float32),
    ],
)

_tc_first = pl.pallas_call(
    _tc_first_body,
    out_shape=jax.ShapeDtypeStruct((2, _N, _HH), jnp.float32),
)

_tc_h = pl.pallas_call(
    _tc_h_body,
    grid=(_NG,),
    in_specs=[
        pl.BlockSpec((_BLKP, _H), lambda i: (i, 0)),
        pl.BlockSpec((2, 2, _BLKP, _H), lambda i: (0, 0, i, 0)),
        pl.BlockSpec((2, _BLKP, _H), lambda i: (0, i, 0)),
        pl.BlockSpec((2, _H), lambda i: (0, 0)),
        pl.BlockSpec((2, 2, _H, _H), lambda i: (0, 0, 0, 0)),
    ],
    out_specs=[
        pl.BlockSpec((2, _BLKP, _H), lambda i: (0, i, 0)),
        pl.BlockSpec((2, _BLKP, _H), lambda i: (0, i, 0)),
    ],
    out_shape=[
        jax.ShapeDtypeStruct((2, _NP, _H), jnp.float32),
        jax.ShapeDtypeStruct((2, _NP, _H), jnp.float32),
    ],
)

_tc_pool = pl.pallas_call(
    _tc_pool_body,
    grid=(_NG,),
    in_specs=[
        pl.BlockSpec((2, _BLKP, _H), lambda i: (0, i, 0)),
        pl.BlockSpec((2, _H), lambda i: (0, 0)),
        pl.BlockSpec((1, 1), lambda i: (0, 0)),
    ],
    out_specs=pl.BlockSpec((1, _H), lambda i: (0, 0)),
    out_shape=jax.ShapeDtypeStruct((1, _H), jnp.float32),
    scratch_shapes=[
        pltpu.VMEM((2, _H), jnp.float32),
        pltpu.SMEM((1,), jnp.float32),
    ],
)


@jax.jit
def kernel(x, edge_index, W1, b1, W2, b2, W3, b3, gW1, gb1, gW2, gb2, gW3, gb3):
    src = edge_index[0]
    dst = edge_index[1]
    npad = _EPAD - _E
    pad_dst = _N + (jnp.arange(npad, dtype=jnp.int32) % (_NPAD - _N))
    pad_src = jnp.arange(npad, dtype=jnp.int32) % _N
    src3 = jnp.concatenate([src, pad_src])
    dst3 = jnp.concatenate([dst, pad_dst])
    src3 = src3.reshape(_NW, _CH, _CHUNK)
    dst3 = dst3.reshape(_NW, _CH, _CHUNK)

    ones16 = jnp.ones((_CHUNK, 16), jnp.float32)
    zeros16 = jnp.zeros((_RPT, 16), jnp.float32)
    zerosH = jnp.zeros((_RPT, _HH), jnp.float32)

    degp = _sc_deg(dst3, ones16, zeros16)
    dn, db = _tc_dinv(degp)
    dpk = db.reshape(_NP, _H)

    y1 = _tc_first(dn, x, W1)

    def pack_vec(v):
        vh = v.reshape(2, _HH)
        return jnp.concatenate([vh, vh], axis=1)

    def pack_w(wn):
        blocks = wn.reshape(2, _HH, 2, _HH).transpose(0, 2, 1, 3)
        z = jnp.zeros((2, 2, _HH, _HH), jnp.float32)
        top = jnp.concatenate([blocks, z], axis=-1)
        bot = jnp.concatenate([z, blocks], axis=-1)
        return jnp.concatenate([top, bot], axis=-2)

    bs = jnp.stack([pack_vec(b1), pack_vec(b2), pack_vec(b3)])
    gws = jnp.stack([pack_vec(gW1[:, 0]), pack_vec(gW2[:, 0]),
                     pack_vec(gW3[:, 0])])
    gbs = jnp.stack([gb1.reshape(1, 1), gb2.reshape(1, 1), gb3.reshape(1, 1)])
    wns = jnp.stack([pack_w(W2), pack_w(W3), pack_w(W3)])

    def step(carry, params):
        y_sc, h_prev = carry
        b_pk, gw_pk, gb, d = params
        p = _sc_agg(y_sc, src3, dst3, zerosH)
        r_prev = _tc_pool(h_prev, gw_pk, gb)
        p128 = p.reshape(_NC, 2, _NPP, _H)
        h, y_next = _tc_h(dpk, p128, y_sc.reshape(2, _NP, _H), b_pk, d)
        return (y_next.reshape(2, _N, _HH), h), r_prev

    gws_shift = jnp.concatenate([gws[0:1], gws[:2]])
    gbs_shift = jnp.concatenate([gbs[0:1], gbs[:2]])
    (_, h_last), rs = lax.scan(
        step, (y1, jnp.zeros((2, _NP, _H), jnp.float32)),
        (bs, gws_shift, gbs_shift, wns))
    r3 = _tc_pool(h_last, gws[2], gbs[2])
    return jnp.concatenate([rs[1], rs[2], r3], axis=1)


# speedup vs baseline: 1.1755x; 1.0686x over previous
import functools

import jax
import jax.numpy as jnp
from jax import lax
from jax.experimental import pallas as pl
from jax.experimental.pallas import tpu as pltpu
from jax.experimental.pallas import tpu_sc as plsc

_N = 10000
_H = 128
_HH = 64
_E = 320000

_NC = 2
_NS = 16
_NW = _NC * _NS
_CHUNK = 128
_CH = 80
_EPW = _CHUNK * _CH
_EPAD = _NW * _EPW
_NPAD = 10240
_RPT = _NPAD // _NS

_mesh = plsc.VectorSubcoreMesh(core_axis_name="c", subcore_axis_name="s")

_sc_params = pltpu.CompilerParams(use_tc_tiling_on_sc=False)


@functools.partial(
    pl.kernel,
    out_type=jax.ShapeDtypeStruct((_NC, _NPAD, 16), jnp.float32),
    mesh=_mesh,
    scratch_types=[
        pltpu.VMEM((_CH, _CHUNK), jnp.int32),
        pltpu.VMEM((_CHUNK, 16), jnp.float32),
        pltpu.VMEM_SHARED((_NPAD, 16), jnp.float32),
    ],
    compiler_params=_sc_params,
)
def _sc_deg(dst_hbm, ones_hbm, zeros_hbm, out_hbm, dst_v, ones_v, acc):
    c = lax.axis_index("c")
    s = lax.axis_index("s")
    w = c * _NS + s
    pltpu.sync_copy(dst_hbm.at[w], dst_v)
    pltpu.sync_copy(ones_hbm, ones_v)
    pltpu.sync_copy(zeros_hbm, acc.at[pl.ds(s * _RPT, _RPT)])
    plsc.subcore_barrier()

    @pl.loop(0, _CH)
    def _(j):
        pltpu.sync_copy(ones_v, acc.at[dst_v.at[j]], add=True)

    plsc.subcore_barrier()
    pltpu.sync_copy(acc.at[pl.ds(s * _RPT, _RPT)],
                    out_hbm.at[c, pl.ds(s * _RPT, _RPT)])


@functools.partial(
    pl.kernel,
    out_type=jax.ShapeDtypeStruct((_NC, 2, _NPAD, _HH), jnp.float32),
    mesh=_mesh,
    scratch_types=[
        pltpu.VMEM((_CH, _CHUNK), jnp.int32),
        pltpu.VMEM((_CH, _CHUNK), jnp.int32),
        pltpu.VMEM((4, _CHUNK, _HH), jnp.float32),
        pltpu.VMEM_SHARED((_NPAD, _HH), jnp.float32),
        [pltpu.SemaphoreType.DMA] * 4,
        [pltpu.SemaphoreType.DMA] * 4,
    ],
    compiler_params=_sc_params,
)
def _sc_agg(y_hbm, src_hbm, dst_hbm, zeros_hbm, out_hbm,
            src_v, dst_v, bufs, acc, gsems, ssems):
    c = lax.axis_index("c")
    s = lax.axis_index("s")
    w = c * _NS + s
    pltpu.sync_copy(src_hbm.at[w], src_v)
    pltpu.sync_copy(dst_hbm.at[w], dst_v)

    for h in range(2):
        yh = y_hbm.at[h]
        pltpu.sync_copy(zeros_hbm, acc.at[pl.ds(s * _RPT, _RPT)])
        plsc.subcore_barrier()

        pltpu.async_copy(yh.at[src_v.at[0]], bufs.at[0], gsems[0])
        pltpu.async_copy(yh.at[src_v.at[1]], bufs.at[1], gsems[1])

        @pl.loop(0, _CH, step=4)
        def _(j):
            for b in range(4):
                m = j + b
                b2 = (b + 2) % 4
                pltpu.make_async_copy(yh.at[src_v.at[0]], bufs.at[b],
                                      gsems[b]).wait()
                pltpu.async_copy(bufs.at[b], acc.at[dst_v.at[m]], ssems[b],
                                 add=True)

                @pl.when(m + 2 < _CH)
                def _():
                    @pl.when(m - 2 >= 0)
                    def _():
                        pltpu.make_async_copy(
                            bufs.at[b2], acc.at[dst_v.at[0]], ssems[b2]).wait()

                    pltpu.async_copy(yh.at[src_v.at[m + 2]], bufs.at[b2],
                                     gsems[b2])

        for b in range(4):
            pltpu.make_async_copy(bufs.at[b], acc.at[dst_v.at[0]],
                                  ssems[b]).wait()

        plsc.subcore_barrier()
        pltpu.sync_copy(acc.at[pl.ds(s * _RPT, _RPT)],
                        out_hbm.at[c, h, pl.ds(s * _RPT, _RPT)])


_NP = _N // 2
_NPP = _NPAD // 2
_BLKP = 1000
_NG = _NP // _BLKP


def _tc_dinv_body(degp_ref, dn_ref, db_ref):
    deg = degp_ref[0, : _N, 0:1] + degp_ref[1, : _N, 0:1] + 1.0
    dn = lax.rsqrt(deg)
    dn_ref[...] = dn
    db_ref[...] = jnp.broadcast_to(dn, (_N, _HH))


def _tc_first_body(dn_ref, x_ref, w_ref, y_ref):
    y = jnp.dot(x_ref[...], w_ref[...]) * dn_ref[...]
    y_ref[0] = y[:, :_HH]
    y_ref[1] = y[:, _HH:]


def _tc_h_body(dpk_ref, p_ref, y_ref, bpk_ref, d_ref, h_ref, yn_ref):
    dpk = dpk_ref[...]
    h0 = jnp.maximum(
        (p_ref[0, 0] + p_ref[1, 0] + y_ref[0]) * dpk + bpk_ref[0:1, :], 0.0)
    h1 = jnp.maximum(
        (p_ref[0, 1] + p_ref[1, 1] + y_ref[1]) * dpk + bpk_ref[1:2, :], 0.0)
    h_ref[0] = h0
    h_ref[1] = h1
    yn_ref[0] = (jnp.dot(h0, d_ref[0, 0]) + jnp.dot(h1, d_ref[1, 0])) * dpk
    yn_ref[1] = (jnp.dot(h0, d_ref[0, 1]) + jnp.dot(h1, d_ref[1, 1])) * dpk


def _tc_pool_body(h_ref, gwpk_ref, gb_ref, r_ref, v_ref, s_ref):
    i = pl.program_id(0)
    h0 = h_ref[0]
    h1 = h_ref[1]
    t = h0 * gwpk_ref[0:1, :] + h1 * gwpk_ref[1:2, :]
    g_lo = jnp.sum(t[:, :_HH], axis=1, keepdims=True) + gb_ref[0, 0]
    g_hi = jnp.sum(t[:, _HH:], axis=1, keepdims=True) + gb_ref[0, 0]
    e_lo = jnp.exp(1.0 / (1.0 + jnp.exp(-g_lo)))
    e_hi = jnp.exp(1.0 / (1.0 + jnp.exp(-g_hi)))
    epk = jnp.concatenate([jnp.broadcast_to(e_lo, (_BLKP, _HH)),
                           jnp.broadcast_to(e_hi, (_BLKP, _HH))], axis=1)
    pv = jnp.concatenate([jnp.sum(epk * h0, axis=0, keepdims=True),
                          jnp.sum(epk * h1, axis=0, keepdims=True)], axis=0)
    ps = jnp.sum(e_lo) + jnp.sum(e_hi)

    @pl.when(i == 0)
    def _():
        v_ref[...] = pv
        s_ref[0] = ps

    @pl.when(i > 0)
    def _():
        v_ref[...] += pv
        s_ref[0] += ps

    @pl.when(i == _NG - 1)
    def _():
        v = v_ref[...] / s_ref[0]
        r_ref[...] = jnp.concatenate(
            [v[0:1, :_HH] + v[0:1, _HH:], v[1:2, :_HH] + v[1:2, _HH:]],
            axis=1)


_tc_dinv = pl.pallas_call(
    _tc_dinv_body,
    out_shape=[
        jax.ShapeDtypeStruct((_N, 1), jnp.float32),
        jax.ShapeDtypeStruct((_N, _HH), jnp.float32),
    ],
)

_tc_first = pl.pallas_call(
    _tc_first_body,
    out_shape=jax.ShapeDtypeStruct((2, _N, _HH), jnp.float32),
)

_tc_h = pl.pallas_call(
    _tc_h_body,
    grid=(_NG,),
    in_specs=[
        pl.BlockSpec((_BLKP, _H), lambda i: (i, 0)),
        pl.BlockSpec((2, 2, _BLKP, _H), lambda i: (0, 0, i, 0)),
        pl.BlockSpec((2, _BLKP, _H), lambda i: (0, i, 0)),
        pl.BlockSpec((2, _H), lambda i: (0, 0)),
        pl.BlockSpec((2, 2, _H, _H), lambda i: (0, 0, 0, 0)),
    ],
    out_specs=[
        pl.BlockSpec((2, _BLKP, _H), lambda i: (0, i, 0)),
        pl.BlockSpec((2, _BLKP, _H), lambda i: (0, i, 0)),
    ],
    out_shape=[
        jax.ShapeDtypeStruct((2, _NP, _H), jnp.float32),
        jax.ShapeDtypeStruct((2, _NP, _H), jnp.float32),
    ],
)

_tc_pool = pl.pallas_call(
    _tc_pool_body,
    grid=(_NG,),
    in_specs=[
        pl.BlockSpec((2, _BLKP, _H), lambda i: (0, i, 0)),
        pl.BlockSpec((2, _H), lambda i: (0, 0)),
        pl.BlockSpec((1, 1), lambda i: (0, 0)),
    ],
    out_specs=pl.BlockSpec((1, _H), lambda i: (0, 0)),
    out_shape=jax.ShapeDtypeStruct((1, _H), jnp.float32),
    scratch_shapes=[
        pltpu.VMEM((2, _H), jnp.float32),
        pltpu.SMEM((1,), jnp.float32),
    ],
)


@jax.jit
def kernel(x, edge_index, W1, b1, W2, b2, W3, b3, gW1, gb1, gW2, gb2, gW3, gb3):
    src = edge_index[0]
    dst = edge_index[1]
    npad = _EPAD - _E
    pad_dst = _N + (jnp.arange(npad, dtype=jnp.int32) % (_NPAD - _N))
    pad_src = jnp.arange(npad, dtype=jnp.int32) % _N
    src3 = jnp.concatenate([src, pad_src])
    dst3 = jnp.concatenate([dst, pad_dst])
    src3 = src3.reshape(_NW, _CH, _CHUNK)
    dst3 = dst3.reshape(_NW, _CH, _CHUNK)

    ones16 = jnp.ones((_CHUNK, 16), jnp.float32)
    zeros16 = jnp.zeros((_RPT, 16), jnp.float32)
    zerosH = jnp.zeros((_RPT, _HH), jnp.float32)

    degp = _sc_deg(dst3, ones16, zeros16)
    dn, db = _tc_dinv(degp)
    dpk = db.reshape(_NP, _H)

    y1 = _tc_first(dn, x, W1)

    def pack_vec(v):
        vh = v.reshape(2, _HH)
        return jnp.concatenate([vh, vh], axis=1)

    def pack_w(wn):
        blocks = wn.reshape(2, _HH, 2, _HH).transpose(0, 2, 1, 3)
        z = jnp.zeros((2, 2, _HH, _HH), jnp.float32)
        top = jnp.concatenate([blocks, z], axis=-1)
        bot = jnp.concatenate([z, blocks], axis=-1)
        return jnp.concatenate([top, bot], axis=-2)

    bs = jnp.stack([pack_vec(b1), pack_vec(b2), pack_vec(b3)])
    gws = jnp.stack([pack_vec(gW1[:, 0]), pack_vec(gW2[:, 0]),
                     pack_vec(gW3[:, 0])])
    gbs = jnp.stack([gb1.reshape(1, 1), gb2.reshape(1, 1), gb3.reshape(1, 1)])
    wns = jnp.stack([pack_w(W2), pack_w(W3), pack_w(W3)])

    def step(carry, params):
        y_pk, h_prev = carry
        b_pk, gw_pk, gb, d = params
        p = _sc_agg(y_pk.reshape(2, _N, _HH), src3, dst3, zerosH)
        r_prev = _tc_pool(h_prev, gw_pk, gb)
        p128 = p.reshape(_NC, 2, _NPP, _H)
        h, y_next = _tc_h(dpk, p128, y_pk, b_pk, d)
        return (y_next, h), r_prev

    gws_shift = jnp.concatenate([gws[0:1], gws[:2]])
    gbs_shift = jnp.concatenate([gbs[0:1], gbs[:2]])
    (_, h_last), rs = lax.scan(
        step, (y1.reshape(2, _NP, _H), jnp.zeros((2, _NP, _H), jnp.float32)),
        (bs, gws_shift, gbs_shift, wns))
    r3 = _tc_pool(h_last, gws[2], gbs[2])
    return jnp.concatenate([rs[1], rs[2], r3], axis=1)
